# Initial kernel scaffold; baseline (speedup 1.0000x reference)
import functools

import jax
import jax.numpy as jnp
from jax import lax
from jax.experimental import pallas as pl
from jax.experimental.pallas import tpu as pltpu
from jax.experimental.pallas import tpu_sc as plsc

_NUM_USERS = 500000
_LANES = 16


def _make_body(batch, emb, nw):
    bpw = batch // nw
    ch = 128
    nch = bpw // ch

    def body(id_hbm, eu_hbm, ei_hbm, out_hbm, ids_v, uidx_v, iidx_v, rows_v,
             out_v, gsem):
        nc = lax.axis_size("c")
        wid = lax.axis_index("s") * nc + lax.axis_index("c")
        base = wid * bpw

        pltpu.sync_copy(id_hbm.at[pl.ds(base, bpw)], ids_v)

        for k in range(bpw // _LANES):
            idv = ids_v[pl.ds(k * _LANES, _LANES)]
            m = idv < _NUM_USERS
            c, o = k // (ch // _LANES), (k % (ch // _LANES)) * _LANES
            uidx_v[c, pl.ds(o, _LANES)] = jnp.where(m, idv, 0)
            iidx_v[c, pl.ds(o, _LANES)] = jnp.where(m, 0, idv - _NUM_USERS)

        copies = []
        for c in range(nch):
            copies.append(pltpu.async_copy(
                eu_hbm.at[uidx_v.at[c]], rows_v.at[pl.ds(c * ch, ch)], gsem))
            copies.append(pltpu.async_copy(
                ei_hbm.at[iidx_v.at[c]], rows_v.at[pl.ds(bpw + c * ch, ch)],
                gsem))
        for cp in copies:
            cp.wait()

        def rows4(i, carry):
            for u in range(4):
                r = i * 4 + u
                sel = jnp.where(ids_v[r] < _NUM_USERS, r, r + bpw)
                for cc in range(emb // _LANES):
                    out_v[r, pl.ds(cc * _LANES, _LANES)] = (
                        rows_v[sel, pl.ds(cc * _LANES, _LANES)])
            return carry

        lax.fori_loop(0, bpw // 4, rows4, 0)

        pltpu.sync_copy(out_v, out_hbm.at[pl.ds(base, bpw)])

    return body, bpw, ch, nch


def kernel(id, e_user, e_item):
    batch = id.shape[0]
    emb = e_user.shape[1]
    info = plsc.get_sparse_core_info()
    nw = info.num_cores * info.num_subcores
    body, bpw, ch, nch = _make_body(batch, emb, nw)
    mesh = plsc.VectorSubcoreMesh(core_axis_name="c", subcore_axis_name="s")
    f = pl.kernel(
        body,
        out_type=jax.ShapeDtypeStruct((batch, emb), jnp.float32),
        mesh=mesh,
        scratch_types=[
            pltpu.VMEM((bpw,), jnp.int32),
            pltpu.VMEM((nch, ch), jnp.int32),
            pltpu.VMEM((nch, ch), jnp.int32),
            pltpu.VMEM((2 * bpw, emb), jnp.float32),
            pltpu.VMEM((bpw, emb), jnp.float32),
            pltpu.SemaphoreType.DMA,
        ],
    )
    return f(id, e_user, e_item)


# speedup vs baseline: 1.2162x; 1.2162x over previous
import functools

import jax
import jax.numpy as jnp
from jax import lax
from jax.experimental import pallas as pl
from jax.experimental.pallas import tpu as pltpu
from jax.experimental.pallas import tpu_sc as plsc

_NUM_USERS = 500000
_LANES = 16


def _make_body(batch, emb, nw):
    bpw = batch // nw
    ch = 128
    nch = bpw // ch

    def body(id_hbm, eu_hbm, ei_hbm, out_hbm, ids_v, uidx_v, iidx_v, rows_v,
             out_v, gsem):
        nc = lax.axis_size("c")
        wid = lax.axis_index("s") * nc + lax.axis_index("c")
        base = wid * bpw

        pltpu.sync_copy(id_hbm.at[pl.ds(base, bpw)], ids_v)

        for k in range(bpw // _LANES):
            idv = ids_v[pl.ds(k * _LANES, _LANES)]
            m = idv < _NUM_USERS
            c, o = k // (ch // _LANES), (k % (ch // _LANES)) * _LANES
            uidx_v[c, pl.ds(o, _LANES)] = jnp.where(m, idv, 0)
            iidx_v[c, pl.ds(o, _LANES)] = jnp.where(m, 0, idv - _NUM_USERS)

        copies = []
        for c in range(nch):
            copies.append(pltpu.async_copy(
                eu_hbm.at[uidx_v.at[c]], rows_v.at[pl.ds(c * ch, ch)], gsem))
            copies.append(pltpu.async_copy(
                ei_hbm.at[iidx_v.at[c]], rows_v.at[pl.ds(bpw + c * ch, ch)],
                gsem))
        for cp in copies:
            cp.wait()

        def rows16(g, carry):
            gbase = g * _LANES
            idv = ids_v[pl.ds(gbase, _LANES)]
            for u in range(_LANES):
                r = gbase + u
                sel = jnp.where(idv[u] < _NUM_USERS, r, r + bpw)
                for cc in range(emb // _LANES):
                    out_v[r, pl.ds(cc * _LANES, _LANES)] = (
                        rows_v[sel, pl.ds(cc * _LANES, _LANES)])
            return carry

        lax.fori_loop(0, bpw // _LANES, rows16, 0)

        pltpu.sync_copy(out_v, out_hbm.at[pl.ds(base, bpw)])

    return body, bpw, ch, nch


def kernel(id, e_user, e_item):
    batch = id.shape[0]
    emb = e_user.shape[1]
    info = plsc.get_sparse_core_info()
    nw = info.num_cores * info.num_subcores
    body, bpw, ch, nch = _make_body(batch, emb, nw)
    mesh = plsc.VectorSubcoreMesh(core_axis_name="c", subcore_axis_name="s")
    f = pl.kernel(
        body,
        out_type=jax.ShapeDtypeStruct((batch, emb), jnp.float32),
        mesh=mesh,
        compiler_params=pltpu.CompilerParams(use_tc_tiling_on_sc=False),
        scratch_types=[
            pltpu.VMEM((bpw,), jnp.int32),
            pltpu.VMEM((nch, ch), jnp.int32),
            pltpu.VMEM((nch, ch), jnp.int32),
            pltpu.VMEM((2 * bpw, emb), jnp.float32),
            pltpu.VMEM((bpw, emb), jnp.float32),
            pltpu.SemaphoreType.DMA,
        ],
    )
    return f(id, e_user, e_item)
